# Initial kernel scaffold; baseline (speedup 1.0000x reference)
import jax
import jax.numpy as jnp
from jax.experimental import pallas as pl


def kernel(X, table, W, b):
    raise NotImplementedError("write your pallas kernel here")


# speedup vs baseline: 1.1404x; 1.1404x over previous
import functools

import jax
import jax.numpy as jnp
from jax import lax
from jax.experimental import pallas as pl
from jax.experimental.pallas import tpu as pltpu
from jax.experimental.pallas import tpu_sc as plsc

B = 64
S = 200
P = 128
NC = 2
NS = 16
NW = NC * NS
ROWS_PER_W = B // NW
L = 16
CHUNKS = ((0, 128), (128, 72))


def _sc_pooled_sums(X, table):
    mesh = plsc.VectorSubcoreMesh(core_axis_name="c", subcore_axis_name="s")

    @functools.partial(
        pl.kernel,
        mesh=mesh,
        out_type=jax.ShapeDtypeStruct((B, P), jnp.float32),
        scratch_types=[
            pltpu.VMEM((S,), jnp.int32),
            pltpu.VMEM((S, P), jnp.float32),
            pltpu.VMEM((P,), jnp.float32),
            pltpu.SemaphoreType.DMA,
        ],
    )
    def k(x_hbm, table_hbm, out_hbm, idx_v, rows_v, acc_v, sem):
        wid = lax.axis_index("s") * NC + lax.axis_index("c")
        for j in range(ROWS_PER_W):
            b_row = wid * ROWS_PER_W + j
            pltpu.sync_copy(x_hbm.at[b_row], idx_v)
            for off, n in CHUNKS:
                pltpu.async_copy(
                    table_hbm.at[idx_v.at[pl.ds(off, n)]],
                    rows_v.at[pl.ds(off, n)],
                    sem,
                ).wait()

            def body(r, acc):
                return tuple(
                    acc[c] + rows_v[r, pl.ds(c * L, L)] for c in range(P // L)
                )

            zero = jnp.zeros((L,), jnp.float32)
            acc = lax.fori_loop(0, S, body, (zero,) * (P // L))
            for c in range(P // L):
                acc_v[pl.ds(c * L, L)] = acc[c]
            pltpu.sync_copy(acc_v, out_hbm.at[b_row])

    return k(X, table)


def _tc_head(pooled, W, b2):
    def body(pooled_ref, w_ref, b_ref, o_ref):
        z = jnp.sum(pooled_ref[...] * w_ref[...], axis=1, keepdims=True)
        z = z * (1.0 / S) + b_ref[0, 0]
        o_ref[...] = jax.nn.sigmoid(z)

    return pl.pallas_call(
        body,
        out_shape=jax.ShapeDtypeStruct((B, 1), jnp.float32),
    )(pooled, W, b2)


def kernel(X, table, W, b):
    pooled = _sc_pooled_sums(X, table)
    out = _tc_head(pooled, W, b.reshape(1, 1))
    return out.reshape(B)


# speedup vs baseline: 1.2246x; 1.0738x over previous
import functools

import jax
import jax.numpy as jnp
from jax import lax
from jax.experimental import pallas as pl
from jax.experimental.pallas import tpu as pltpu
from jax.experimental.pallas import tpu_sc as plsc

B = 64
S = 200
P = 128
NC = 2
NS = 16
NW = NC * NS
ROWS_PER_W = B // NW
L = 16
CHUNKS = ((0, 128), (128, 72))


def _sc_pooled_sums(X, table):
    mesh = plsc.VectorSubcoreMesh(core_axis_name="c", subcore_axis_name="s")
    SW = S * ROWS_PER_W

    @functools.partial(
        pl.kernel,
        mesh=mesh,
        out_type=jax.ShapeDtypeStruct((B, P), jnp.float32),
        scratch_types=[
            pltpu.VMEM((SW,), jnp.int32),
            pltpu.VMEM((SW, P), jnp.float32),
            pltpu.VMEM((ROWS_PER_W, P), jnp.float32),
            pltpu.SemaphoreType.DMA,
            pltpu.SemaphoreType.DMA,
        ],
    )
    def k(x_hbm, table_hbm, out_hbm, idx_v, rows_v, acc_v, sem0, sem1):
        wid = lax.axis_index("s") * NC + lax.axis_index("c")
        pltpu.sync_copy(x_hbm.at[pl.ds(wid * SW, SW)], idx_v)
        sems = (sem0, sem1)
        copies = []
        for j in range(ROWS_PER_W):
            for off, n in CHUNKS:
                copies.append(
                    pltpu.async_copy(
                        table_hbm.at[idx_v.at[pl.ds(j * S + off, n)]],
                        rows_v.at[pl.ds(j * S + off, n)],
                        sems[j],
                    )
                )
        for j in range(ROWS_PER_W):
            for c_idx in range(len(CHUNKS)):
                copies[j * len(CHUNKS) + c_idx].wait()

            def body(r, acc, base=j * S):
                return tuple(
                    acc[c] + rows_v[base + r, pl.ds(c * L, L)]
                    for c in range(P // L)
                )

            zero = jnp.zeros((L,), jnp.float32)
            acc = lax.fori_loop(0, S, body, (zero,) * (P // L), unroll=8)
            for c in range(P // L):
                acc_v[j, pl.ds(c * L, L)] = acc[c]
        pltpu.sync_copy(acc_v, out_hbm.at[pl.ds(wid * ROWS_PER_W, ROWS_PER_W)])

    return k(X.reshape(-1), table)


def _tc_head(pooled, W, b2):
    def body(pooled_ref, w_ref, b_ref, o_ref):
        z = jnp.sum(pooled_ref[...] * w_ref[...], axis=1, keepdims=True)
        z = z * (1.0 / S) + b_ref[0, 0]
        o_ref[...] = jax.nn.sigmoid(z)

    return pl.pallas_call(
        body,
        out_shape=jax.ShapeDtypeStruct((B, 1), jnp.float32),
    )(pooled, W, b2)


def kernel(X, table, W, b):
    pooled = _sc_pooled_sums(X, table)
    out = _tc_head(pooled, W, b.reshape(1, 1))
    return out.reshape(B)
